# Initial kernel scaffold; baseline (speedup 1.0000x reference)
import jax
import jax.numpy as jnp
from jax.experimental import pallas as pl


def kernel(x, embed_weight):
    raise NotImplementedError("write your pallas kernel here")


# speedup vs baseline: 28.2165x; 28.2165x over previous
import functools

import jax
import jax.numpy as jnp
from jax import lax
from jax.experimental import pallas as pl
from jax.experimental.pallas import tpu as pltpu
from jax.experimental.pallas import tpu_sc as plsc

NC, NS = 2, 16
NW = NC * NS
D = 16

CHUNK = 128


def _encoder_call(xp, embed_weight, B, HW, n_chunks):
    BPW = B // NW
    inv_n = 1.0 / HW

    mesh = plsc.VectorSubcoreMesh(
        core_axis_name="c", subcore_axis_name="s",
        num_cores=NC, num_subcores=NS)

    @functools.partial(
        pl.kernel,
        out_type=jax.ShapeDtypeStruct((B * D,), jnp.float32),
        mesh=mesh,
        scratch_types=[
            pltpu.VMEM((n_chunks, CHUNK), jnp.int32),
            pltpu.VMEM((n_chunks * CHUNK, D), jnp.float32),
            pltpu.VMEM((BPW * D,), jnp.float32),
            pltpu.SemaphoreType.DMA,
        ],
        compiler_params=pltpu.CompilerParams(use_tc_tiling_on_sc=False),
    )
    def enc(x_hbm, tab_hbm, out_hbm, idx_v, rows_v, out_v, sem):
        wid = lax.axis_index("s") * NC + lax.axis_index("c")
        base = wid * BPW

        def body(i, _):
            pltpu.sync_copy(x_hbm.at[base + i], idx_v)
            descs = []
            for j in range(n_chunks):
                descs.append(pltpu.async_copy(
                    tab_hbm.at[idx_v.at[j]],
                    rows_v.at[pl.ds(j * CHUNK, CHUNK)], sem))
            for dsc in descs:
                dsc.wait()

            def red(k, acc):
                return acc + rows_v[k]
            acc = lax.fori_loop(0, HW, red, jnp.zeros((D,), jnp.float32),
                                unroll=8)
            out_v[pl.ds(i * D, D)] = acc * inv_n
            return 0

        lax.fori_loop(0, BPW, body, 0)
        pltpu.sync_copy(out_v, out_hbm.at[pl.ds(base * D, BPW * D)])

    return enc(xp, embed_weight).reshape(B, D)


def kernel(x, embed_weight):
    if x.ndim == 4 and x.shape[1] == 1:
        x = jnp.squeeze(x, axis=1)
    B = x.shape[0]
    HW = x.shape[1] * x.shape[2]
    n_chunks = (HW + CHUNK - 1) // CHUNK
    pad = n_chunks * CHUNK - HW
    xf = x.reshape(B, HW).astype(jnp.int32)
    xp = jnp.pad(xf, ((0, 0), (0, pad))).reshape(B, n_chunks, CHUNK)
    return _encoder_call(xp, embed_weight, B, HW, n_chunks)


# speedup vs baseline: 28.3824x; 1.0059x over previous
import functools

import jax
import jax.numpy as jnp
from jax import lax
from jax.experimental import pallas as pl
from jax.experimental.pallas import tpu as pltpu
from jax.experimental.pallas import tpu_sc as plsc

NC, NS = 2, 16
NW = NC * NS
D = 16

CHUNK = 128


def _encoder_call(xp, embed_weight, B, HW, n_chunks):
    BPW = B // NW
    assert BPW % 2 == 0
    inv_n = 1.0 / HW
    n4 = HW // 4
    tail = HW - 4 * n4

    mesh = plsc.VectorSubcoreMesh(
        core_axis_name="c", subcore_axis_name="s",
        num_cores=NC, num_subcores=NS)

    @functools.partial(
        pl.kernel,
        out_type=jax.ShapeDtypeStruct((B * D,), jnp.float32),
        mesh=mesh,
        scratch_types=[
            pltpu.VMEM((n_chunks, CHUNK), jnp.int32),
            pltpu.VMEM((n_chunks, CHUNK), jnp.int32),
            pltpu.VMEM((n_chunks * CHUNK, D), jnp.float32),
            pltpu.VMEM((n_chunks * CHUNK, D), jnp.float32),
            pltpu.VMEM((BPW * D,), jnp.float32),
            pltpu.SemaphoreType.DMA,
            pltpu.SemaphoreType.DMA,
            pltpu.SemaphoreType.DMA,
            pltpu.SemaphoreType.DMA,
        ],
        compiler_params=pltpu.CompilerParams(use_tc_tiling_on_sc=False),
    )
    def enc(x_hbm, tab_hbm, out_hbm, idx0, idx1, rows0, rows1, out_v,
            sg0, sg1, si0, si1):
        wid = lax.axis_index("s") * NC + lax.axis_index("c")
        base = wid * BPW

        def fire(idx_v, rows_v, sem):
            for j in range(n_chunks):
                pltpu.async_copy(
                    tab_hbm.at[idx_v.at[j]],
                    rows_v.at[pl.ds(j * CHUNK, CHUNK)], sem)

        def drain(rows_v, sem):
            for j in range(n_chunks):
                pltpu.make_async_copy(
                    tab_hbm.at[pl.ds(0, CHUNK)],
                    rows_v.at[pl.ds(j * CHUNK, CHUNK)], sem).wait()

        def reduce_store(rows_v, r):
            z = jnp.zeros((D,), jnp.float32)

            def red(k, accs):
                a0, a1, a2, a3 = accs
                b = 4 * k
                return (a0 + rows_v[b], a1 + rows_v[b + 1],
                        a2 + rows_v[b + 2], a3 + rows_v[b + 3])
            a0, a1, a2, a3 = lax.fori_loop(0, n4, red, (z, z, z, z),
                                           unroll=4)
            acc = (a0 + a1) + (a2 + a3)
            for k in range(tail):
                acc = acc + rows_v[4 * n4 + k]
            out_v[pl.ds(r * D, D)] = acc * inv_n

        def phase(t, r, idx_v, rows_v, sg, si):
            drain(rows_v, sg)

            @pl.when(t < BPW // 2 - 1)
            def _():
                pltpu.async_copy(x_hbm.at[base + r + 2], idx_v, si)
            reduce_store(rows_v, r)

            @pl.when(t < BPW // 2 - 1)
            def _():
                pltpu.make_async_copy(x_hbm.at[base + r + 2], idx_v,
                                      si).wait()
                fire(idx_v, rows_v, sg)

        pltpu.sync_copy(x_hbm.at[base], idx0)
        fire(idx0, rows0, sg0)
        pltpu.sync_copy(x_hbm.at[base + 1], idx1)
        fire(idx1, rows1, sg1)

        def body(t, _):
            phase(t, 2 * t, idx0, rows0, sg0, si0)
            phase(t, 2 * t + 1, idx1, rows1, sg1, si1)
            return 0

        lax.fori_loop(0, BPW // 2, body, 0)
        pltpu.sync_copy(out_v, out_hbm.at[pl.ds(base * D, BPW * D)])

    return enc(xp, embed_weight).reshape(B, D)


def kernel(x, embed_weight):
    if x.ndim == 4 and x.shape[1] == 1:
        x = jnp.squeeze(x, axis=1)
    B = x.shape[0]
    HW = x.shape[1] * x.shape[2]
    n_chunks = (HW + CHUNK - 1) // CHUNK
    pad = n_chunks * CHUNK - HW
    xf = x.reshape(B, HW).astype(jnp.int32)
    xp = jnp.pad(xf, ((0, 0), (0, pad))).reshape(B, n_chunks, CHUNK)
    return _encoder_call(xp, embed_weight, B, HW, n_chunks)


# speedup vs baseline: 77.9730x; 2.7472x over previous
import functools

import jax
import jax.numpy as jnp
from jax import lax
from jax.experimental import pallas as pl
from jax.experimental.pallas import tpu as pltpu
from jax.experimental.pallas import tpu_sc as plsc

NC, NS = 2, 16
NW = NC * NS
D = 16

CHUNK = 128
N_PIECE = 4


def _encoder_call(xp, embed_weight, B, HW, n_chunks):
    BPW = B // NW
    inv_n = 1.0 / HW
    V = embed_weight.shape[0]
    rows_per_tile = ((V + NS - 1) // NS + 7) // 8 * 8

    assert n_chunks % N_PIECE == 0
    cpp = n_chunks // N_PIECE
    piece = cpp * CHUNK
    real = [max(0, min(HW - p * piece, piece)) for p in range(N_PIECE)]

    mesh = plsc.VectorSubcoreMesh(
        core_axis_name="c", subcore_axis_name="s",
        num_cores=NC, num_subcores=NS)

    @functools.partial(
        pl.kernel,
        out_type=jax.ShapeDtypeStruct((B * D,), jnp.float32),
        mesh=mesh,
        scratch_types=[
            pltpu.VMEM((n_chunks, CHUNK), jnp.int32),
            pltpu.VMEM((n_chunks, CHUNK), jnp.int32),
            pltpu.VMEM((piece, D), jnp.float32),
            pltpu.VMEM((piece, D), jnp.float32),
            pltpu.VMEM((BPW * D,), jnp.float32),
            pltpu.VMEM_SHARED((V, D), jnp.float32),
            pltpu.SemaphoreType.DMA,
            pltpu.SemaphoreType.DMA,
            pltpu.SemaphoreType.DMA,
        ],
        compiler_params=pltpu.CompilerParams(use_tc_tiling_on_sc=False),
    )
    def enc(x_hbm, tab_hbm, out_hbm, idx0, idx1, buf0, buf1, out_v,
            tab_sp, sg0, sg1, si):
        wid = lax.axis_index("s") * NC + lax.axis_index("c")
        base = wid * BPW
        sid = lax.axis_index("s")

        lo = jnp.minimum(sid * rows_per_tile, V - rows_per_tile)
        pltpu.sync_copy(tab_hbm.at[pl.ds(lo, rows_per_tile)],
                        tab_sp.at[pl.ds(lo, rows_per_tile)])
        plsc.subcore_barrier()

        bufs = (buf0, buf1)
        sems = (sg0, sg1)

        def fire(idx_v, p):
            for jj in range(cpp):
                pltpu.async_copy(
                    tab_sp.at[idx_v.at[p * cpp + jj]],
                    bufs[p % 2].at[pl.ds(jj * CHUNK, CHUNK)],
                    sems[p % 2])

        def drain(p):
            pltpu.make_async_copy(
                tab_hbm.at[pl.ds(0, piece)], bufs[p % 2],
                sems[p % 2]).wait()

        def red(p, accs):
            buf = bufs[p % 2]
            n4 = real[p] // 4

            def step(k, a):
                b = 4 * k
                return (a[0] + buf[b], a[1] + buf[b + 1],
                        a[2] + buf[b + 2], a[3] + buf[b + 3])
            accs = lax.fori_loop(0, n4, step, accs, unroll=4)
            for k in range(real[p] - 4 * n4):
                accs = (accs[0] + buf[4 * n4 + k],) + accs[1:]
            return accs

        def make_row(idx_v, idx_next):
            def row(r, _):
                @pl.when(r + 1 < BPW)
                def _():
                    pltpu.async_copy(x_hbm.at[base + r + 1], idx_next, si)
                fire(idx_v, 0)
                fire(idx_v, 1)
                z = jnp.zeros((D,), jnp.float32)
                accs = (z, z, z, z)
                for p in range(N_PIECE):
                    drain(p)
                    if p + 2 < N_PIECE:
                        fire(idx_v, p + 2)
                    accs = red(p, accs)
                acc = (accs[0] + accs[1]) + (accs[2] + accs[3])
                out_v[pl.ds(r * D, D)] = acc * inv_n

                @pl.when(r + 1 < BPW)
                def _():
                    pltpu.make_async_copy(x_hbm.at[base + r + 1], idx_next,
                                          si).wait()
                return 0
            return row

        row_even = make_row(idx0, idx1)
        row_odd = make_row(idx1, idx0)

        pltpu.sync_copy(x_hbm.at[base], idx0)

        def pair(t, _):
            row_even(2 * t, 0)
            row_odd(2 * t + 1, 0)
            return 0

        lax.fori_loop(0, BPW // 2, pair, 0)
        pltpu.sync_copy(out_v, out_hbm.at[pl.ds(base * D, BPW * D)])

    return enc(xp, embed_weight).reshape(B, D)


def kernel(x, embed_weight):
    if x.ndim == 4 and x.shape[1] == 1:
        x = jnp.squeeze(x, axis=1)
    B = x.shape[0]
    HW = x.shape[1] * x.shape[2]
    n_chunks = (HW + CHUNK - 1) // CHUNK
    n_chunks = ((n_chunks + N_PIECE - 1) // N_PIECE) * N_PIECE
    pad = n_chunks * CHUNK - HW
    xf = x.reshape(B, HW).astype(jnp.int32)
    xp = jnp.pad(xf, ((0, 0), (0, pad))).reshape(B, n_chunks, CHUNK)
    return _encoder_call(xp, embed_weight, B, HW, n_chunks)


# speedup vs baseline: 78.1213x; 1.0019x over previous
import functools

import jax
import jax.numpy as jnp
from jax import lax
from jax.experimental import pallas as pl
from jax.experimental.pallas import tpu as pltpu
from jax.experimental.pallas import tpu_sc as plsc

NC, NS = 2, 16
NW = NC * NS
D = 16

CHUNK = 640
N_PIECE = 4


def _encoder_call(xp, embed_weight, B, HW, n_chunks):
    BPW = B // NW
    inv_n = 1.0 / HW
    V = embed_weight.shape[0]
    rows_per_tile = ((V + NS - 1) // NS + 7) // 8 * 8

    assert n_chunks % N_PIECE == 0
    cpp = n_chunks // N_PIECE
    piece = cpp * CHUNK
    real = [max(0, min(HW - p * piece, piece)) for p in range(N_PIECE)]

    mesh = plsc.VectorSubcoreMesh(
        core_axis_name="c", subcore_axis_name="s",
        num_cores=NC, num_subcores=NS)

    @functools.partial(
        pl.kernel,
        out_type=jax.ShapeDtypeStruct((B * D,), jnp.float32),
        mesh=mesh,
        scratch_types=[
            pltpu.VMEM((n_chunks, CHUNK), jnp.int32),
            pltpu.VMEM((n_chunks, CHUNK), jnp.int32),
            pltpu.VMEM((piece, D), jnp.float32),
            pltpu.VMEM((piece, D), jnp.float32),
            pltpu.VMEM((BPW * D,), jnp.float32),
            pltpu.VMEM_SHARED((V, D), jnp.float32),
            pltpu.SemaphoreType.DMA,
            pltpu.SemaphoreType.DMA,
            pltpu.SemaphoreType.DMA,
        ],
        compiler_params=pltpu.CompilerParams(use_tc_tiling_on_sc=False),
    )
    def enc(x_hbm, tab_hbm, out_hbm, idx0, idx1, buf0, buf1, out_v,
            tab_sp, sg0, sg1, si):
        wid = lax.axis_index("s") * NC + lax.axis_index("c")
        base = wid * BPW
        sid = lax.axis_index("s")

        lo = jnp.minimum(sid * rows_per_tile, V - rows_per_tile)
        pltpu.sync_copy(tab_hbm.at[pl.ds(lo, rows_per_tile)],
                        tab_sp.at[pl.ds(lo, rows_per_tile)])
        plsc.subcore_barrier()

        bufs = (buf0, buf1)
        sems = (sg0, sg1)

        def fire(idx_v, p):
            for jj in range(cpp):
                pltpu.async_copy(
                    tab_sp.at[idx_v.at[p * cpp + jj]],
                    bufs[p % 2].at[pl.ds(jj * CHUNK, CHUNK)],
                    sems[p % 2])

        def drain(p):
            pltpu.make_async_copy(
                tab_hbm.at[pl.ds(0, piece)], bufs[p % 2],
                sems[p % 2]).wait()

        def red(p, accs):
            buf = bufs[p % 2]
            n4 = real[p] // 4

            def step(k, a):
                b = 4 * k
                return (a[0] + buf[b], a[1] + buf[b + 1],
                        a[2] + buf[b + 2], a[3] + buf[b + 3])
            accs = lax.fori_loop(0, n4, step, accs, unroll=4)
            for k in range(real[p] - 4 * n4):
                accs = (accs[0] + buf[4 * n4 + k],) + accs[1:]
            return accs

        def make_row(idx_v, idx_next):
            def row(r, _):
                @pl.when(r + 1 < BPW)
                def _():
                    pltpu.async_copy(x_hbm.at[base + r + 1], idx_next, si)
                fire(idx_v, 0)
                fire(idx_v, 1)
                z = jnp.zeros((D,), jnp.float32)
                accs = (z, z, z, z)
                for p in range(N_PIECE):
                    drain(p)
                    if p + 2 < N_PIECE:
                        fire(idx_v, p + 2)
                    accs = red(p, accs)
                acc = (accs[0] + accs[1]) + (accs[2] + accs[3])
                out_v[pl.ds(r * D, D)] = acc * inv_n

                @pl.when(r + 1 < BPW)
                def _():
                    pltpu.make_async_copy(x_hbm.at[base + r + 1], idx_next,
                                          si).wait()
                return 0
            return row

        row_even = make_row(idx0, idx1)
        row_odd = make_row(idx1, idx0)

        pltpu.sync_copy(x_hbm.at[base], idx0)

        def pair(t, _):
            row_even(2 * t, 0)
            row_odd(2 * t + 1, 0)
            return 0

        lax.fori_loop(0, BPW // 2, pair, 0)
        pltpu.sync_copy(out_v, out_hbm.at[pl.ds(base * D, BPW * D)])

    return enc(xp, embed_weight).reshape(B, D)


def kernel(x, embed_weight):
    if x.ndim == 4 and x.shape[1] == 1:
        x = jnp.squeeze(x, axis=1)
    B = x.shape[0]
    HW = x.shape[1] * x.shape[2]
    n_chunks = (HW + CHUNK - 1) // CHUNK
    n_chunks = ((n_chunks + N_PIECE - 1) // N_PIECE) * N_PIECE
    pad = n_chunks * CHUNK - HW
    xf = x.reshape(B, HW).astype(jnp.int32)
    xp = jnp.pad(xf, ((0, 0), (0, pad))).reshape(B, n_chunks, CHUNK)
    return _encoder_call(xp, embed_weight, B, HW, n_chunks)
